# baseline (speedup 1.0000x reference)
import functools

import jax
import jax.numpy as jnp
from jax import lax
from jax.experimental import pallas as pl
from jax.experimental.pallas import tpu as pltpu, tpu_sc as plsc

N_NODES = 10000
N_EDGES = 320000
D = 128

NC = 2
NS = 16
NW = NC * NS
CHUNK = 128
N_PAD = 10240
E_PAD = 327680
CH_PER_W = E_PAD // (NW * CHUNK)
ROWS_PER_TILE = N_PAD // NS
DUMMY = N_NODES

_mesh = plsc.VectorSubcoreMesh(
    core_axis_name="c", subcore_axis_name="s", num_cores=NC, num_subcores=NS)


@functools.partial(
    pl.kernel,
    out_type=jax.ShapeDtypeStruct((NC, N_PAD, D), jnp.float32),
    mesh=_mesh,
    scratch_types=[
        pltpu.VMEM((CH_PER_W, CHUNK), jnp.int32),
        pltpu.VMEM((CHUNK, D), jnp.float32),
        pltpu.VMEM_SHARED((N_PAD, D), jnp.float32),
    ],
)
def _sc_degrees(eidx_hbm, ones_hbm, zrows_hbm, h_out, idx_v, ones_v, hist):
    c = lax.axis_index("c")
    s = lax.axis_index("s")
    base = s * ROWS_PER_TILE

    pltpu.sync_copy(ones_hbm, ones_v)
    pltpu.sync_copy(zrows_hbm, hist.at[pl.ds(base, ROWS_PER_TILE)])
    plsc.subcore_barrier()

    for half in range(2):
        pltpu.sync_copy(eidx_hbm.at[c, s, half], idx_v)

        def add(j, carry):
            pltpu.sync_copy(ones_v, hist.at[idx_v.at[j]], add=True)
            return carry
        lax.fori_loop(0, CH_PER_W, add, None)

    plsc.subcore_barrier()
    pltpu.sync_copy(hist.at[pl.ds(base, ROWS_PER_TILE)],
                    h_out.at[c, pl.ds(base, ROWS_PER_TILE)])


AC = 64
TOT_CH = E_PAD // AC
G2 = 16
NB = 4
N0 = 48
N1 = TOT_CH // NS - N0


@functools.partial(
    pl.kernel,
    out_type=jax.ShapeDtypeStruct((NC, N_PAD, D), jnp.float32),
    mesh=_mesh,
    scratch_types=[
        pltpu.VMEM((G2, AC), jnp.int32),
        pltpu.VMEM((G2, AC), jnp.int32),
        [pltpu.VMEM((AC, D), jnp.float32)] * NB,
        pltpu.VMEM_SHARED((N_PAD, D), jnp.float32),
        [pltpu.SemaphoreType.DMA] * NB,
        [pltpu.SemaphoreType.DMA] * NB,
    ],
)
def _sc_edge_agg(pre_hbm, src_hbm, dst_hbm, zrows_hbm, acc_out,
                 src_idx, dst_idx, rows, acc, sem_g, sem_s):
    c = lax.axis_index("c")
    s = lax.axis_index("s")
    base = s * ROWS_PER_TILE
    my_ng = jnp.where(c == 0, N0 // G2, N1 // G2)
    ch_base = jnp.where(c == 0, s * N0, NS * N0 + s * N1)

    pltpu.sync_copy(zrows_hbm, acc.at[pl.ds(base, ROWS_PER_TILE)])
    plsc.subcore_barrier()

    def gather(j, l):
        pltpu.async_copy(pre_hbm.at[src_idx.at[j]], rows[l], sem_g[l])

    def gather_wait(j, l):
        pltpu.make_async_copy(pre_hbm.at[src_idx.at[j]], rows[l],
                              sem_g[l]).wait()

    def scatter(j, l):
        pltpu.async_copy(rows[l], acc.at[dst_idx.at[j]], sem_s[l], add=True)

    def scatter_wait(j, l):
        pltpu.make_async_copy(rows[l], acc.at[dst_idx.at[j]],
                              sem_s[l]).wait()

    def group(g, carry):
        pltpu.sync_copy(src_hbm.at[pl.ds(ch_base + g * G2, G2)], src_idx)
        pltpu.sync_copy(dst_hbm.at[pl.ds(ch_base + g * G2, G2)], dst_idx)
        gather(0, 0)
        gather(1, 1)

        def quad(i, carry2):
            for l in range(NB):
                j = NB * i + l
                jn = j + 2
                ln = (l + 2) % NB

                @pl.when(jnp.logical_and(jn >= NB, jn < G2))
                def _():
                    scatter_wait(jn - NB, ln)

                @pl.when(jn < G2)
                def _():
                    gather(jn, ln)

                gather_wait(j, l)
                scatter(j, l)
            return carry2
        lax.fori_loop(0, G2 // NB, quad, None)
        for t in range(NB):
            scatter_wait(G2 - NB + t, (G2 - NB + t) % NB)
        return carry
    lax.fori_loop(0, my_ng, group, None)

    plsc.subcore_barrier()
    pltpu.sync_copy(acc.at[pl.ds(base, ROWS_PER_TILE)],
                    acc_out.at[c, pl.ds(base, ROWS_PER_TILE)])


_BM = 1024


def _norm(deg):
    return lax.rsqrt(jnp.maximum(deg, 1.0))


def _tc_pre1_body(x_ref, w_ref, deg_ref, o_ref):
    o_ref[...] = (jnp.dot(x_ref[...], w_ref[...],
                          preferred_element_type=jnp.float32)
                  * _norm(deg_ref[...]))


_tc_pre1 = pl.pallas_call(
    _tc_pre1_body,
    grid=(N_PAD // _BM,),
    in_specs=[
        pl.BlockSpec((_BM, D), lambda i: (i, 0)),
        pl.BlockSpec((D, D), lambda i: (0, 0)),
        pl.BlockSpec((_BM, 1), lambda i: (i, 0)),
    ],
    out_specs=pl.BlockSpec((_BM, D), lambda i: (i, 0)),
    out_shape=jax.ShapeDtypeStruct((N_PAD, D), jnp.float32),
)


def _tc_mid_body(acc_ref, indeg_ref, outdeg_ref, b_ref, w_ref, o_ref):
    agg = acc_ref[0] + acc_ref[1]
    h = jax.nn.relu(agg * _norm(indeg_ref[...]) + b_ref[...])
    o_ref[...] = (jnp.dot(h, w_ref[...], preferred_element_type=jnp.float32)
                  * _norm(outdeg_ref[...]))


_tc_mid = pl.pallas_call(
    _tc_mid_body,
    grid=(N_PAD // _BM,),
    in_specs=[
        pl.BlockSpec((NC, _BM, D), lambda i: (0, i, 0)),
        pl.BlockSpec((_BM, 1), lambda i: (i, 0)),
        pl.BlockSpec((_BM, 1), lambda i: (i, 0)),
        pl.BlockSpec((1, D), lambda i: (0, 0)),
        pl.BlockSpec((D, D), lambda i: (0, 0)),
    ],
    out_specs=pl.BlockSpec((_BM, D), lambda i: (i, 0)),
    out_shape=jax.ShapeDtypeStruct((N_PAD, D), jnp.float32),
)

_BF = 1000


def _tc_final_body(acc_ref, indeg_ref, b_ref, o_ref):
    agg = acc_ref[0] + acc_ref[1]
    h = jax.nn.relu(agg * _norm(indeg_ref[...]) + b_ref[...])
    part = jnp.sum(h, axis=0, keepdims=True)

    @pl.when(pl.program_id(0) == 0)
    def _():
        o_ref[...] = part

    @pl.when(pl.program_id(0) > 0)
    def _():
        o_ref[...] += part


_tc_final = pl.pallas_call(
    _tc_final_body,
    grid=(N_NODES // _BF,),
    in_specs=[
        pl.BlockSpec((NC, _BF, D), lambda i: (0, i, 0)),
        pl.BlockSpec((_BF, 1), lambda i: (i, 0)),
        pl.BlockSpec((1, D), lambda i: (0, 0)),
    ],
    out_specs=pl.BlockSpec((1, D), lambda i: (0, 0)),
    out_shape=jax.ShapeDtypeStruct((1, D), jnp.float32),
)


def kernel(x, edge_index, W1, b1, W2, b2, W3, b3):
    src = edge_index[0].astype(jnp.int32)
    dst = edge_index[1].astype(jnp.int32)
    pad = jnp.full((E_PAD - N_EDGES,), DUMMY, jnp.int32)
    src_r = jnp.concatenate([src, pad]).reshape(TOT_CH, AC)
    dst_r = jnp.concatenate([dst, pad]).reshape(TOT_CH, AC)
    x_pad = jnp.pad(x, ((0, N_PAD - N_NODES), (0, 0)))

    ones_rows = jnp.ones((CHUNK, D), jnp.float32)
    zrows = jnp.zeros((ROWS_PER_TILE, D), jnp.float32)
    eidx = jnp.stack([src_r, dst_r]).reshape(NC, NS, 2, CH_PER_W, CHUNK)

    hists = _sc_degrees(eidx, ones_rows, zrows)
    outdeg = hists[0, :, :1]
    indeg = hists[1, :, :1]

    b1r = b1.reshape(1, D)
    b2r = b2.reshape(1, D)
    b3r = b3.reshape(1, D)

    pre = _tc_pre1(x_pad, W1, outdeg)
    acc = _sc_edge_agg(pre, src_r, dst_r, zrows)
    pre = _tc_mid(acc, indeg, outdeg, b1r, W2)
    acc = _sc_edge_agg(pre, src_r, dst_r, zrows)
    pre = _tc_mid(acc, indeg, outdeg, b2r, W3)
    acc = _sc_edge_agg(pre, src_r, dst_r, zrows)
    return _tc_final(acc, indeg, b3r)


# speedup vs baseline: 1.5227x; 1.5227x over previous
import functools

import jax
import jax.numpy as jnp
from jax import lax
from jax.experimental import pallas as pl
from jax.experimental.pallas import tpu as pltpu, tpu_sc as plsc

N_NODES = 10000
N_EDGES = 320000
D = 128

NC = 2
NS = 16
NW = NC * NS
CHUNK = 128
N_PAD = 10240
E_PAD = 327680
CH_PER_W = E_PAD // (NW * CHUNK)
ROWS_PER_TILE = N_PAD // NS
DUMMY = N_NODES

_mesh = plsc.VectorSubcoreMesh(
    core_axis_name="c", subcore_axis_name="s", num_cores=NC, num_subcores=NS)


@functools.partial(
    pl.kernel,
    out_type=jax.ShapeDtypeStruct((NC, N_PAD, D), jnp.float32),
    mesh=_mesh,
    scratch_types=[
        pltpu.VMEM((CH_PER_W, CHUNK), jnp.int32),
        pltpu.VMEM((CHUNK, D), jnp.float32),
        pltpu.VMEM_SHARED((N_PAD, D), jnp.float32),
    ],
)
def _sc_degrees(eidx_hbm, ones_hbm, zrows_hbm, h_out, idx_v, ones_v, hist):
    c = lax.axis_index("c")
    s = lax.axis_index("s")
    base = s * ROWS_PER_TILE

    pltpu.sync_copy(ones_hbm, ones_v)
    pltpu.sync_copy(zrows_hbm, hist.at[pl.ds(base, ROWS_PER_TILE)])
    plsc.subcore_barrier()

    for half in range(2):
        pltpu.sync_copy(eidx_hbm.at[c, s, half], idx_v)

        def add(j, carry):
            pltpu.sync_copy(ones_v, hist.at[idx_v.at[j]], add=True)
            return carry
        lax.fori_loop(0, CH_PER_W, add, None)

    plsc.subcore_barrier()
    pltpu.sync_copy(hist.at[pl.ds(base, ROWS_PER_TILE)],
                    h_out.at[c, pl.ds(base, ROWS_PER_TILE)])


AC = 64
TOT_CH = E_PAD // AC
G2 = 16
NB = 4
N0 = 288
N1 = TOT_CH // NS - N0


@functools.partial(
    pl.kernel,
    out_type=jax.ShapeDtypeStruct((NC, N_PAD, D), jnp.float32),
    mesh=_mesh,
    scratch_types=[
        pltpu.VMEM((G2, AC), jnp.int32),
        pltpu.VMEM((G2, AC), jnp.int32),
        [pltpu.VMEM((AC, D), jnp.float32)] * NB,
        pltpu.VMEM_SHARED((N_PAD, D), jnp.float32),
        [pltpu.SemaphoreType.DMA] * NB,
        [pltpu.SemaphoreType.DMA] * NB,
    ],
)
def _sc_edge_agg(pre_hbm, src_hbm, dst_hbm, zrows_hbm, acc_out,
                 src_idx, dst_idx, rows, acc, sem_g, sem_s):
    c = lax.axis_index("c")
    s = lax.axis_index("s")
    base = s * ROWS_PER_TILE
    my_ng = jnp.where(c == 0, N0 // G2, N1 // G2)
    ch_base = jnp.where(c == 0, s * N0, NS * N0 + s * N1)

    pltpu.sync_copy(zrows_hbm, acc.at[pl.ds(base, ROWS_PER_TILE)])
    plsc.subcore_barrier()

    def gather(j, l):
        pltpu.async_copy(pre_hbm.at[src_idx.at[j]], rows[l], sem_g[l])

    def gather_wait(j, l):
        pltpu.make_async_copy(pre_hbm.at[src_idx.at[j]], rows[l],
                              sem_g[l]).wait()

    def scatter(j, l):
        pltpu.async_copy(rows[l], acc.at[dst_idx.at[j]], sem_s[l], add=True)

    def scatter_wait(j, l):
        pltpu.make_async_copy(rows[l], acc.at[dst_idx.at[j]],
                              sem_s[l]).wait()

    def group(g, carry):
        pltpu.sync_copy(src_hbm.at[pl.ds(ch_base + g * G2, G2)], src_idx)
        pltpu.sync_copy(dst_hbm.at[pl.ds(ch_base + g * G2, G2)], dst_idx)
        gather(0, 0)
        gather(1, 1)

        def quad(i, carry2):
            for l in range(NB):
                j = NB * i + l
                jn = j + 2
                ln = (l + 2) % NB

                @pl.when(jnp.logical_and(jn >= NB, jn < G2))
                def _():
                    scatter_wait(jn - NB, ln)

                @pl.when(jn < G2)
                def _():
                    gather(jn, ln)

                gather_wait(j, l)
                scatter(j, l)
            return carry2
        lax.fori_loop(0, G2 // NB, quad, None)
        for t in range(NB):
            scatter_wait(G2 - NB + t, (G2 - NB + t) % NB)
        return carry
    lax.fori_loop(0, my_ng, group, None)

    plsc.subcore_barrier()
    pltpu.sync_copy(acc.at[pl.ds(base, ROWS_PER_TILE)],
                    acc_out.at[c, pl.ds(base, ROWS_PER_TILE)])


_BM = 1024


def _norm(deg):
    return lax.rsqrt(jnp.maximum(deg, 1.0))


def _tc_pre1_body(x_ref, w_ref, deg_ref, o_ref):
    o_ref[...] = (jnp.dot(x_ref[...], w_ref[...],
                          preferred_element_type=jnp.float32)
                  * _norm(deg_ref[...]))


_tc_pre1 = pl.pallas_call(
    _tc_pre1_body,
    grid=(N_PAD // _BM,),
    in_specs=[
        pl.BlockSpec((_BM, D), lambda i: (i, 0)),
        pl.BlockSpec((D, D), lambda i: (0, 0)),
        pl.BlockSpec((_BM, 1), lambda i: (i, 0)),
    ],
    out_specs=pl.BlockSpec((_BM, D), lambda i: (i, 0)),
    out_shape=jax.ShapeDtypeStruct((N_PAD, D), jnp.float32),
)


def _tc_mid_body(acc_ref, indeg_ref, outdeg_ref, b_ref, w_ref, o_ref):
    agg = acc_ref[0] + acc_ref[1]
    h = jax.nn.relu(agg * _norm(indeg_ref[...]) + b_ref[...])
    o_ref[...] = (jnp.dot(h, w_ref[...], preferred_element_type=jnp.float32)
                  * _norm(outdeg_ref[...]))


_tc_mid = pl.pallas_call(
    _tc_mid_body,
    grid=(N_PAD // _BM,),
    in_specs=[
        pl.BlockSpec((NC, _BM, D), lambda i: (0, i, 0)),
        pl.BlockSpec((_BM, 1), lambda i: (i, 0)),
        pl.BlockSpec((_BM, 1), lambda i: (i, 0)),
        pl.BlockSpec((1, D), lambda i: (0, 0)),
        pl.BlockSpec((D, D), lambda i: (0, 0)),
    ],
    out_specs=pl.BlockSpec((_BM, D), lambda i: (i, 0)),
    out_shape=jax.ShapeDtypeStruct((N_PAD, D), jnp.float32),
)

_BF = 1000


def _tc_final_body(acc_ref, indeg_ref, b_ref, o_ref):
    agg = acc_ref[0] + acc_ref[1]
    h = jax.nn.relu(agg * _norm(indeg_ref[...]) + b_ref[...])
    part = jnp.sum(h, axis=0, keepdims=True)

    @pl.when(pl.program_id(0) == 0)
    def _():
        o_ref[...] = part

    @pl.when(pl.program_id(0) > 0)
    def _():
        o_ref[...] += part


_tc_final = pl.pallas_call(
    _tc_final_body,
    grid=(N_NODES // _BF,),
    in_specs=[
        pl.BlockSpec((NC, _BF, D), lambda i: (0, i, 0)),
        pl.BlockSpec((_BF, 1), lambda i: (i, 0)),
        pl.BlockSpec((1, D), lambda i: (0, 0)),
    ],
    out_specs=pl.BlockSpec((1, D), lambda i: (0, 0)),
    out_shape=jax.ShapeDtypeStruct((1, D), jnp.float32),
)


def kernel(x, edge_index, W1, b1, W2, b2, W3, b3):
    src = edge_index[0].astype(jnp.int32)
    dst = edge_index[1].astype(jnp.int32)
    pad = jnp.full((E_PAD - N_EDGES,), DUMMY, jnp.int32)
    src_r = jnp.concatenate([src, pad]).reshape(TOT_CH, AC)
    dst_r = jnp.concatenate([dst, pad]).reshape(TOT_CH, AC)
    x_pad = jnp.pad(x, ((0, N_PAD - N_NODES), (0, 0)))

    ones_rows = jnp.ones((CHUNK, D), jnp.float32)
    zrows = jnp.zeros((ROWS_PER_TILE, D), jnp.float32)
    eidx = jnp.stack([src_r, dst_r]).reshape(NC, NS, 2, CH_PER_W, CHUNK)

    hists = _sc_degrees(eidx, ones_rows, zrows)
    outdeg = hists[0, :, :1]
    indeg = hists[1, :, :1]

    b1r = b1.reshape(1, D)
    b2r = b2.reshape(1, D)
    b3r = b3.reshape(1, D)

    pre = _tc_pre1(x_pad, W1, outdeg)
    acc = _sc_edge_agg(pre, src_r, dst_r, zrows)
    pre = _tc_mid(acc, indeg, outdeg, b1r, W2)
    acc = _sc_edge_agg(pre, src_r, dst_r, zrows)
    pre = _tc_mid(acc, indeg, outdeg, b2r, W3)
    acc = _sc_edge_agg(pre, src_r, dst_r, zrows)
    return _tc_final(acc, indeg, b3r)


# speedup vs baseline: 1.5598x; 1.0244x over previous
import functools

import jax
import jax.numpy as jnp
from jax import lax
from jax.experimental import pallas as pl
from jax.experimental.pallas import tpu as pltpu, tpu_sc as plsc

N_NODES = 10000
N_EDGES = 320000
D = 128

NC = 2
NS = 16
NW = NC * NS
CHUNK = 128
N_PAD = 10240
E_PAD = 327680
CH_PER_W = E_PAD // (NW * CHUNK)
ROWS_PER_TILE = N_PAD // NS
DUMMY = N_NODES

_mesh = plsc.VectorSubcoreMesh(
    core_axis_name="c", subcore_axis_name="s", num_cores=NC, num_subcores=NS)


@functools.partial(
    pl.kernel,
    out_type=jax.ShapeDtypeStruct((NC, N_PAD, D), jnp.float32),
    mesh=_mesh,
    scratch_types=[
        pltpu.VMEM((CH_PER_W, CHUNK), jnp.int32),
        pltpu.VMEM((CHUNK, D), jnp.float32),
        pltpu.VMEM_SHARED((N_PAD, D), jnp.float32),
    ],
)
def _sc_degrees(eidx_hbm, ones_hbm, zrows_hbm, h_out, idx_v, ones_v, hist):
    c = lax.axis_index("c")
    s = lax.axis_index("s")
    base = s * ROWS_PER_TILE

    pltpu.sync_copy(ones_hbm, ones_v)
    pltpu.sync_copy(zrows_hbm, hist.at[pl.ds(base, ROWS_PER_TILE)])
    plsc.subcore_barrier()

    for half in range(2):
        pltpu.sync_copy(eidx_hbm.at[c, s, half], idx_v)

        def add(j, carry):
            pltpu.sync_copy(ones_v, hist.at[idx_v.at[j]], add=True)
            return carry
        lax.fori_loop(0, CH_PER_W, add, None)

    plsc.subcore_barrier()
    pltpu.sync_copy(hist.at[pl.ds(base, ROWS_PER_TILE)],
                    h_out.at[c, pl.ds(base, ROWS_PER_TILE)])


AC = 64
TOT_CH = E_PAD // AC
G2 = 16
NB = 4
N0 = 304
N1 = TOT_CH // NS - N0


@functools.partial(
    pl.kernel,
    out_type=jax.ShapeDtypeStruct((NC, N_PAD, D), jnp.float32),
    mesh=_mesh,
    scratch_types=[
        pltpu.VMEM((G2, AC), jnp.int32),
        pltpu.VMEM((G2, AC), jnp.int32),
        [pltpu.VMEM((AC, D), jnp.float32)] * NB,
        pltpu.VMEM_SHARED((N_PAD, D), jnp.float32),
        [pltpu.SemaphoreType.DMA] * NB,
        [pltpu.SemaphoreType.DMA] * NB,
    ],
)
def _sc_edge_agg(pre_hbm, src_hbm, dst_hbm, zrows_hbm, acc_out,
                 src_idx, dst_idx, rows, acc, sem_g, sem_s):
    c = lax.axis_index("c")
    s = lax.axis_index("s")
    base = s * ROWS_PER_TILE
    my_ng = jnp.where(c == 0, N0 // G2, N1 // G2)
    ch_base = jnp.where(c == 0, s * N0, NS * N0 + s * N1)

    pltpu.sync_copy(zrows_hbm, acc.at[pl.ds(base, ROWS_PER_TILE)])
    plsc.subcore_barrier()

    def gather(j, l):
        pltpu.async_copy(pre_hbm.at[src_idx.at[j]], rows[l], sem_g[l])

    def gather_wait(j, l):
        pltpu.make_async_copy(pre_hbm.at[src_idx.at[j]], rows[l],
                              sem_g[l]).wait()

    def scatter(j, l):
        pltpu.async_copy(rows[l], acc.at[dst_idx.at[j]], sem_s[l], add=True)

    def scatter_wait(j, l):
        pltpu.make_async_copy(rows[l], acc.at[dst_idx.at[j]],
                              sem_s[l]).wait()

    def group(g, carry):
        pltpu.sync_copy(src_hbm.at[pl.ds(ch_base + g * G2, G2)], src_idx)
        pltpu.sync_copy(dst_hbm.at[pl.ds(ch_base + g * G2, G2)], dst_idx)
        gather(0, 0)
        gather(1, 1)

        def quad(i, carry2):
            for l in range(NB):
                j = NB * i + l
                jn = j + 2
                ln = (l + 2) % NB

                @pl.when(jnp.logical_and(jn >= NB, jn < G2))
                def _():
                    scatter_wait(jn - NB, ln)

                @pl.when(jn < G2)
                def _():
                    gather(jn, ln)

                gather_wait(j, l)
                scatter(j, l)
            return carry2
        lax.fori_loop(0, G2 // NB, quad, None)
        for t in range(NB):
            scatter_wait(G2 - NB + t, (G2 - NB + t) % NB)
        return carry
    lax.fori_loop(0, my_ng, group, None)

    plsc.subcore_barrier()
    pltpu.sync_copy(acc.at[pl.ds(base, ROWS_PER_TILE)],
                    acc_out.at[c, pl.ds(base, ROWS_PER_TILE)])


_BM = 1024


def _norm(deg):
    return lax.rsqrt(jnp.maximum(deg, 1.0))


def _tc_pre1_body(x_ref, w_ref, deg_ref, o_ref):
    o_ref[...] = (jnp.dot(x_ref[...], w_ref[...],
                          preferred_element_type=jnp.float32)
                  * _norm(deg_ref[...]))


_tc_pre1 = pl.pallas_call(
    _tc_pre1_body,
    grid=(N_PAD // _BM,),
    in_specs=[
        pl.BlockSpec((_BM, D), lambda i: (i, 0)),
        pl.BlockSpec((D, D), lambda i: (0, 0)),
        pl.BlockSpec((_BM, 1), lambda i: (i, 0)),
    ],
    out_specs=pl.BlockSpec((_BM, D), lambda i: (i, 0)),
    out_shape=jax.ShapeDtypeStruct((N_PAD, D), jnp.float32),
)


def _tc_mid_body(acc_ref, indeg_ref, outdeg_ref, b_ref, w_ref, o_ref):
    agg = acc_ref[0] + acc_ref[1]
    h = jax.nn.relu(agg * _norm(indeg_ref[...]) + b_ref[...])
    o_ref[...] = (jnp.dot(h, w_ref[...], preferred_element_type=jnp.float32)
                  * _norm(outdeg_ref[...]))


_tc_mid = pl.pallas_call(
    _tc_mid_body,
    grid=(N_PAD // _BM,),
    in_specs=[
        pl.BlockSpec((NC, _BM, D), lambda i: (0, i, 0)),
        pl.BlockSpec((_BM, 1), lambda i: (i, 0)),
        pl.BlockSpec((_BM, 1), lambda i: (i, 0)),
        pl.BlockSpec((1, D), lambda i: (0, 0)),
        pl.BlockSpec((D, D), lambda i: (0, 0)),
    ],
    out_specs=pl.BlockSpec((_BM, D), lambda i: (i, 0)),
    out_shape=jax.ShapeDtypeStruct((N_PAD, D), jnp.float32),
)

_BF = 1000


def _tc_final_body(acc_ref, indeg_ref, b_ref, o_ref):
    agg = acc_ref[0] + acc_ref[1]
    h = jax.nn.relu(agg * _norm(indeg_ref[...]) + b_ref[...])
    part = jnp.sum(h, axis=0, keepdims=True)

    @pl.when(pl.program_id(0) == 0)
    def _():
        o_ref[...] = part

    @pl.when(pl.program_id(0) > 0)
    def _():
        o_ref[...] += part


_tc_final = pl.pallas_call(
    _tc_final_body,
    grid=(N_NODES // _BF,),
    in_specs=[
        pl.BlockSpec((NC, _BF, D), lambda i: (0, i, 0)),
        pl.BlockSpec((_BF, 1), lambda i: (i, 0)),
        pl.BlockSpec((1, D), lambda i: (0, 0)),
    ],
    out_specs=pl.BlockSpec((1, D), lambda i: (0, 0)),
    out_shape=jax.ShapeDtypeStruct((1, D), jnp.float32),
)


def kernel(x, edge_index, W1, b1, W2, b2, W3, b3):
    src = edge_index[0].astype(jnp.int32)
    dst = edge_index[1].astype(jnp.int32)
    pad = jnp.full((E_PAD - N_EDGES,), DUMMY, jnp.int32)
    src_r = jnp.concatenate([src, pad]).reshape(TOT_CH, AC)
    dst_r = jnp.concatenate([dst, pad]).reshape(TOT_CH, AC)
    x_pad = jnp.pad(x, ((0, N_PAD - N_NODES), (0, 0)))

    ones_rows = jnp.ones((CHUNK, D), jnp.float32)
    zrows = jnp.zeros((ROWS_PER_TILE, D), jnp.float32)
    eidx = jnp.stack([src_r, dst_r]).reshape(NC, NS, 2, CH_PER_W, CHUNK)

    hists = _sc_degrees(eidx, ones_rows, zrows)
    outdeg = hists[0, :, :1]
    indeg = hists[1, :, :1]

    b1r = b1.reshape(1, D)
    b2r = b2.reshape(1, D)
    b3r = b3.reshape(1, D)

    pre = _tc_pre1(x_pad, W1, outdeg)
    acc = _sc_edge_agg(pre, src_r, dst_r, zrows)
    pre = _tc_mid(acc, indeg, outdeg, b1r, W2)
    acc = _sc_edge_agg(pre, src_r, dst_r, zrows)
    pre = _tc_mid(acc, indeg, outdeg, b2r, W3)
    acc = _sc_edge_agg(pre, src_r, dst_r, zrows)
    return _tc_final(acc, indeg, b3r)
